# baseline (speedup 1.0000x reference)
import functools

import jax
import jax.numpy as jnp
from jax import lax
from jax.experimental import pallas as pl
from jax.experimental.pallas import tpu as pltpu
from jax.experimental.pallas import tpu_sc as plsc

VOCAB = 1000000
EMBED_DIM = 64
BATCH = 4096
SEQ = 200

NC = 2
NS = 16
NW = NC * NS

ROWS_PER_W = BATCH // NW
ROWS_PER_CHUNK = 2
NBUF = 4
N_CHUNKS = ROWS_PER_W // ROWS_PER_CHUNK
N_STEADY = N_CHUNKS // NBUF - 1

SPLITS = ((0, 128), (128, SEQ - 128))


@functools.partial(
    pl.kernel,
    out_type=jax.ShapeDtypeStruct((BATCH, SEQ, EMBED_DIM), jnp.float32),
    mesh=plsc.VectorSubcoreMesh(core_axis_name="c", subcore_axis_name="s"),
    scratch_types=[
        pltpu.VMEM((ROWS_PER_W, SEQ), jnp.int32),
        pltpu.VMEM((NBUF, ROWS_PER_CHUNK, SEQ, EMBED_DIM), jnp.float32),
        [pltpu.SemaphoreType.DMA] * NBUF,
        [pltpu.SemaphoreType.DMA] * NBUF,
    ],
    compiler_params=pltpu.CompilerParams(use_tc_tiling_on_sc=False),
)
def _gather_sc(table_hbm, idx_hbm, out_hbm, idx_all, rows_v, gsems, wsems):
    wid = lax.axis_index("s") * NC + lax.axis_index("c")
    base = wid * ROWS_PER_W

    pltpu.sync_copy(idx_hbm.at[pl.ds(base, ROWS_PER_W)], idx_all)

    def start_gather(c, b):
        for k in range(ROWS_PER_CHUNK):
            r = c * ROWS_PER_CHUNK + k
            for off, n in SPLITS:
                pltpu.async_copy(
                    table_hbm.at[idx_all.at[r, pl.ds(off, n)]],
                    rows_v.at[b, k, pl.ds(off, n)],
                    gsems[b],
                )

    def wait_gather(b):
        pltpu.make_async_copy(
            out_hbm.at[pl.ds(0, ROWS_PER_CHUNK)], rows_v.at[b], gsems[b]
        ).wait()

    def start_write(c, b):
        pltpu.async_copy(
            rows_v.at[b],
            out_hbm.at[pl.ds(base + c * ROWS_PER_CHUNK, ROWS_PER_CHUNK)],
            wsems[b],
        )

    def wait_write(b):
        pltpu.make_async_copy(
            rows_v.at[b], out_hbm.at[pl.ds(0, ROWS_PER_CHUNK)], wsems[b]
        ).wait()

    for b in range(NBUF):
        start_gather(b, b)

    def steady(p, carry):
        c0 = p * NBUF
        for b in range(NBUF):
            wait_gather(b)
            start_write(c0 + b, b)
        for b in range(NBUF):
            wait_write(b)
            start_gather(c0 + NBUF + b, b)
        return carry

    lax.fori_loop(0, N_STEADY, steady, 0)

    for b in range(NBUF):
        wait_gather(b)
        start_write(N_CHUNKS - NBUF + b, b)
    for b in range(NBUF):
        wait_write(b)


def kernel(input, table):
    return _gather_sc(table, input)


# speedup vs baseline: 1.3298x; 1.3298x over previous
import functools

import jax
import jax.numpy as jnp
from jax import lax
from jax.experimental import pallas as pl
from jax.experimental.pallas import tpu as pltpu
from jax.experimental.pallas import tpu_sc as plsc

VOCAB = 1000000
EMBED_DIM = 64
BATCH = 4096
SEQ = 200

NC = 2
NS = 16
NW = NC * NS

ROWS_PER_W = BATCH // NW
ROWS_PER_CHUNK = 2
NBUF = 4
N_CHUNKS = ROWS_PER_W // ROWS_PER_CHUNK
N_STEADY = N_CHUNKS // NBUF - 1

SPLITS = ((0, 128), (128, SEQ - 128))


@functools.partial(
    pl.kernel,
    out_type=jax.ShapeDtypeStruct((BATCH * SEQ, 2 * EMBED_DIM), jnp.float32),
    mesh=plsc.VectorSubcoreMesh(core_axis_name="c", subcore_axis_name="s"),
    scratch_types=[
        pltpu.VMEM((ROWS_PER_W, SEQ), jnp.int32),
        pltpu.VMEM((NBUF, ROWS_PER_CHUNK * SEQ, EMBED_DIM), jnp.float32),
        [pltpu.SemaphoreType.DMA] * NBUF,
        [pltpu.SemaphoreType.DMA] * NBUF,
    ],
    compiler_params=pltpu.CompilerParams(use_tc_tiling_on_sc=False),
)
def _gather_sc(table_hbm, idx_hbm, out_hbm, idx_all, rows_v, gsems, wsems):
    wid = lax.axis_index("s") * NC + lax.axis_index("c")
    base = wid * ROWS_PER_W

    pltpu.sync_copy(idx_hbm.at[pl.ds(base, ROWS_PER_W)], idx_all)

    def start_gather(c, b):
        for k in range(ROWS_PER_CHUNK):
            r = c * ROWS_PER_CHUNK + k
            for off, n in SPLITS:
                pltpu.async_copy(
                    table_hbm.at[idx_all.at[r, pl.ds(off, n)]],
                    rows_v.at[b, pl.ds(k * SEQ + off, n)],
                    gsems[b],
                )

    def wait_gather(b):
        pltpu.make_async_copy(
            out_hbm.at[pl.ds(0, ROWS_PER_CHUNK * SEQ), pl.ds(0, EMBED_DIM)],
            rows_v.at[b],
            gsems[b],
        ).wait()

    def start_write(c, b):
        pltpu.async_copy(
            rows_v.at[b],
            out_hbm.at[
                pl.ds((base + c * ROWS_PER_CHUNK) * SEQ, ROWS_PER_CHUNK * SEQ),
                pl.ds(0, EMBED_DIM),
            ],
            wsems[b],
        )

    def wait_write(b):
        pltpu.make_async_copy(
            rows_v.at[b],
            out_hbm.at[pl.ds(0, ROWS_PER_CHUNK * SEQ), pl.ds(0, EMBED_DIM)],
            wsems[b],
        ).wait()

    for b in range(NBUF):
        start_gather(b, b)

    def steady(p, carry):
        c0 = p * NBUF
        for b in range(NBUF):
            wait_gather(b)
            start_write(c0 + b, b)
        for b in range(NBUF):
            wait_write(b)
            start_gather(c0 + NBUF + b, b)
        return carry

    lax.fori_loop(0, N_STEADY, steady, 0)

    for b in range(NBUF):
        wait_gather(b)
        start_write(N_CHUNKS - NBUF + b, b)
    for b in range(NBUF):
        wait_write(b)


def kernel(input, table):
    out = _gather_sc(table, input)
    return out.reshape(BATCH, SEQ, 2 * EMBED_DIM)[:, :, :EMBED_DIM]
